# baseline (speedup 1.0000x reference)
import functools

import jax
import jax.numpy as jnp
from jax import lax
from jax.experimental import pallas as pl
from jax.experimental.pallas import tpu as pltpu
from jax.experimental.pallas import tpu_sc as plsc

_SLICE_W = 128
_NBUF = 8


@functools.lru_cache(maxsize=None)
def _make_sc_gather(V, D, B):
    info = plsc.get_sparse_core_info()
    NC, NS = info.num_cores, info.num_subcores
    NW = NC * NS
    assert B % (16 * NW) == 0
    b_per_w = B // NW
    n_groups = b_per_w // 16
    mesh = plsc.VectorSubcoreMesh(core_axis_name="c", subcore_axis_name="s")

    @functools.partial(
        pl.kernel,
        mesh=mesh,
        compiler_params=pltpu.CompilerParams(needs_layout_passes=False),
        out_type=jax.ShapeDtypeStruct((B * D,), jnp.float32),
        scratch_types=[
            pltpu.VMEM((b_per_w + 16,), jnp.int32),
            pltpu.VMEM((_NBUF, D, _SLICE_W), jnp.float32),
            pltpu.VMEM((b_per_w * D,), jnp.float32),
        ] + [pltpu.SemaphoreType.DMA] * _NBUF,
    )
    def sc_gather(tableT_hbm, idx_hbm, out_hbm, idx_v, stage_v, rows_v, *sems):
        wid = lax.axis_index("s") * NC + lax.axis_index("c")
        base = wid * b_per_w
        pltpu.sync_copy(idx_hbm.at[pl.ds(base, b_per_w)],
                        idx_v.at[pl.ds(0, b_per_w)])
        iota16 = lax.broadcasted_iota(jnp.int32, (16,), 0)
        n_iter = b_per_w // _NBUF

        def issue(v, b):
            start = pl.multiple_of((v // _SLICE_W) * _SLICE_W, 128)
            pltpu.async_copy(
                tableT_hbm.at[pl.ds(0, D), pl.ds(start, _SLICE_W)],
                stage_v.at[b], sems[b],
            )

        vvec0 = idx_v[pl.ds(0, 16)]
        for b in range(_NBUF):
            issue(vvec0[b], b)

        def step(g, carry):
            vvec = idx_v[pl.ds(g * _NBUF, 16)]
            for b in range(_NBUF):
                pltpu.make_async_copy(
                    tableT_hbm.at[pl.ds(0, D), pl.ds(0, _SLICE_W)],
                    stage_v.at[b], sems[b],
                ).wait()
                v = vvec[b]
                l = v - (v // _SLICE_W) * _SLICE_W
                lanes = iota16 * 0 + l
                for k in range(D // 16):
                    rows16 = iota16 + k * 16
                    col = plsc.load_gather(stage_v.at[b], [rows16, lanes])
                    rows_v[pl.ds(g * _NBUF * D + b * D + k * 16, 16)] = col

                @pl.when(g < n_iter - 1)
                def _():
                    issue(vvec[_NBUF + b], b)

            return carry

        lax.fori_loop(0, n_iter, step, 0)
        pltpu.sync_copy(rows_v, out_hbm.at[pl.ds(base * D, b_per_w * D)])

    return sc_gather


def _mlp_block(x_ref, w1_ref, b1_ref, g_ref, be_ref, mu_ref, var_ref,
               w2_ref, b2_ref, o_ref):
    x = x_ref[...]
    h = jnp.dot(x, w1_ref[...], preferred_element_type=jnp.float32)
    h = jnp.maximum(h + b1_ref[...], 0.0)
    s = g_ref[...] * lax.rsqrt(var_ref[...] + 1e-5)
    t = be_ref[...] - mu_ref[...] * s
    h = h * s + t
    o = jnp.dot(h, w2_ref[...], preferred_element_type=jnp.float32)
    o_ref[...] = jnp.maximum(o + b2_ref[...], 0.0)


def _mlp(x, W1, b1, gamma, beta, mu, var, W2, b2, block_b):
    B, D = x.shape
    H = W1.shape[1]
    grid = (B // block_b,)
    row = lambda v: v.reshape(1, H)
    rep = lambda shape: pl.BlockSpec(shape, lambda i: (0, 0))
    return pl.pallas_call(
        _mlp_block,
        grid=grid,
        in_specs=[
            pl.BlockSpec((block_b, D), lambda i: (i, 0)),
            rep((D, H)), rep((1, H)), rep((1, H)), rep((1, H)),
            rep((1, H)), rep((1, H)), rep((H, H)), rep((1, H)),
        ],
        out_specs=pl.BlockSpec((block_b, H), lambda i: (i, 0)),
        out_shape=jax.ShapeDtypeStruct((B, H), jnp.float32),
    )(x, W1, row(b1), row(gamma), row(beta), row(mu), row(var), W2, row(b2))


def kernel(target_ids, emb_table, W1, b1, gamma, beta, running_mean,
           running_var, W2, b2):
    V, D = emb_table.shape
    B = target_ids.shape[0]
    ids = target_ids.astype(jnp.int32)
    tableT = emb_table.T
    n_split = 2
    Bh = B // n_split
    gather = _make_sc_gather(V, D, Bh)
    outs = []
    for i in range(n_split):
        x = gather(tableT, lax.dynamic_slice(ids, (i * Bh,), (Bh,)))
        outs.append(_mlp(x.reshape(Bh, D), W1, b1, gamma, beta, running_mean,
                         running_var, W2, b2, block_b=2048))
    return jnp.concatenate(outs, axis=0)


# speedup vs baseline: 1.2869x; 1.2869x over previous
import functools

import jax
import jax.numpy as jnp
from jax import lax
from jax.experimental import pallas as pl
from jax.experimental.pallas import tpu as pltpu
from jax.experimental.pallas import tpu_sc as plsc

_NBUF = 7


@functools.lru_cache(maxsize=None)
def _make_sc_gather(V, D, B):
    info = plsc.get_sparse_core_info()
    NC, NS = info.num_cores, info.num_subcores
    NW = NC * NS
    assert B % (16 * NW) == 0 and D % 16 == 0
    b_per_w = B // NW
    n_chunks = b_per_w // 16
    mesh = plsc.VectorSubcoreMesh(core_axis_name="c", subcore_axis_name="s")

    @functools.partial(
        pl.kernel,
        mesh=mesh,
        compiler_params=pltpu.CompilerParams(needs_layout_passes=False),
        out_type=jax.ShapeDtypeStruct((B, 128), jnp.float32),
        scratch_types=[
            pltpu.VMEM((b_per_w + 16,), jnp.int32),
            pltpu.VMEM((b_per_w + 16,), jnp.int32),
            pltpu.VMEM((b_per_w + 16,), jnp.int32),
            pltpu.VMEM((b_per_w // 128, 128), jnp.int32),
            pltpu.VMEM((_NBUF, D, 128), jnp.float32),
            pltpu.VMEM((b_per_w, 128), jnp.float32),
        ] + [pltpu.SemaphoreType.DMA] * (_NBUF + 1),
    )
    def sc_gather(tableT_hbm, ids_hbm, pos_hbm, out_hbm,
                  idx_v, s_v, ucols, pos_v, stage_v, rows_v, *sems):
        wid = lax.axis_index("s") * NC + lax.axis_index("c")
        base = wid * b_per_w
        pltpu.sync_copy(ids_hbm.at[pl.ds(base, b_per_w)],
                        idx_v.at[pl.ds(0, b_per_w)])
        pltpu.sync_copy(pos_hbm.at[pl.ds(wid * (b_per_w // 128),
                                         b_per_w // 128)], pos_v)
        iota16 = lax.broadcasted_iota(jnp.int32, (16,), 0)
        big = jnp.full((16,), jnp.int32(1 << 30))
        s_v[pl.ds(b_per_w, 16)] = big

        def chunk(c, carry):
            vv = idx_v[pl.ds(c * 16, 16)]
            t = vv // 128
            shift_idx = jnp.maximum(iota16 + c * 16 - 1, 0)
            tprev = plsc.load_gather(idx_v, [shift_idx]) // 128
            newf = (t != tprev) | ((iota16 + c * 16) == 0)
            nf = newf.astype(jnp.int32)
            cum = plsc.cumsum(nf)
            s = carry + cum - 1
            s_v[pl.ds(c * 16, 16)] = s
            plsc.store_scatter(ucols, [s], t, mask=newf)
            return carry + cum[15]

        n_unique = lax.fori_loop(0, n_chunks, chunk, 0)
        n_grp = (n_unique + _NBUF - 1) // _NBUF

        def group(g, e0):
            uv = plsc.load_gather(ucols, [iota16 + g * _NBUF])
            for b in range(_NBUF):
                @pl.when(g * _NBUF + b < n_unique)
                def _(b=b):
                    start = pl.multiple_of(uv[b] * 128, 128)
                    pltpu.async_copy(
                        tableT_hbm.at[pl.ds(0, D), pl.ds(start, 128)],
                        stage_v.at[b], sems[b],
                    )
            for b in range(_NBUF):
                @pl.when(g * _NBUF + b < n_unique)
                def _(b=b):
                    pltpu.make_async_copy(
                        tableT_hbm.at[pl.ds(0, D), pl.ds(0, 128)],
                        stage_v.at[b], sems[b],
                    ).wait()
            hi = (g + 1) * _NBUF

            def cond(e):
                se = plsc.load_gather(s_v, [iota16 * 0 + e])[0]
                return (e < b_per_w) & (se < hi)

            def body(e):
                s_e = plsc.load_gather(s_v, [iota16 * 0 + e])[0]
                v_e = plsc.load_gather(idx_v, [iota16 * 0 + e])[0]
                slot = iota16 * 0 + (s_e - g * _NBUF)
                lane = iota16 * 0 + (v_e - (v_e // 128) * 128)
                ev = iota16 * 0 + e
                for k in range(D // 16):
                    col = plsc.load_gather(
                        stage_v, [slot, iota16 + k * 16, lane])
                    plsc.store_scatter(rows_v, [ev, iota16 + k * 16], col)
                return e + 1

            return lax.while_loop(cond, body, e0)

        lax.fori_loop(0, n_grp, group, 0)

        cps = []
        for j in range(b_per_w // 128):
            cps.append(pltpu.async_copy(
                rows_v.at[pl.ds(j * 128, 128)],
                out_hbm.at[pos_v.at[j]], sems[_NBUF]))
        for cp in cps:
            cp.wait()

    return sc_gather


def _mlp_block(d_model, x_ref, w1_ref, b1_ref, g_ref, be_ref, mu_ref,
               var_ref, w2_ref, b2_ref, o_ref):
    x = x_ref[...][:, :d_model]
    h = jnp.dot(x, w1_ref[...], preferred_element_type=jnp.float32)
    h = jnp.maximum(h + b1_ref[...], 0.0)
    s = g_ref[...] * lax.rsqrt(var_ref[...] + 1e-5)
    t = be_ref[...] - mu_ref[...] * s
    h = h * s + t
    o = jnp.dot(h, w2_ref[...], preferred_element_type=jnp.float32)
    o_ref[...] = jnp.maximum(o + b2_ref[...], 0.0)


def _mlp(x, D, W1, b1, gamma, beta, mu, var, W2, b2, block_b):
    B, Dw = x.shape
    H = W1.shape[1]
    grid = (B // block_b,)
    row = lambda v: v.reshape(1, H)
    rep = lambda shape: pl.BlockSpec(shape, lambda i: (0, 0))
    return pl.pallas_call(
        functools.partial(_mlp_block, D),
        grid=grid,
        in_specs=[
            pl.BlockSpec((block_b, Dw), lambda i: (i, 0)),
            rep((D, H)), rep((1, H)), rep((1, H)), rep((1, H)),
            rep((1, H)), rep((1, H)), rep((H, H)), rep((1, H)),
        ],
        out_specs=pl.BlockSpec((block_b, H), lambda i: (i, 0)),
        out_shape=jax.ShapeDtypeStruct((B, H), jnp.float32),
    )(x, W1, row(b1), row(gamma), row(beta), row(mu), row(var), W2, row(b2))


def kernel(target_ids, emb_table, W1, b1, gamma, beta, running_mean,
           running_var, W2, b2):
    V, D = emb_table.shape
    B = target_ids.shape[0]
    ids = target_ids.astype(jnp.int32)
    ids_s, pos = lax.sort_key_val(ids, lax.iota(jnp.int32, B))
    pos2 = pos.reshape(B // 128, 128)
    xw = _make_sc_gather(V, D, B)(emb_table.T, ids_s, pos2)
    return _mlp(xw, D, W1, b1, gamma, beta, running_mean, running_var, W2, b2,
                block_b=2048)


# speedup vs baseline: 1.6042x; 1.2465x over previous
import functools

import jax
import jax.numpy as jnp
from jax import lax
from jax.experimental import pallas as pl
from jax.experimental.pallas import tpu as pltpu
from jax.experimental.pallas import tpu_sc as plsc

_NBUF = 5


@functools.lru_cache(maxsize=None)
def _make_sc_gather(V, D, B):
    info = plsc.get_sparse_core_info()
    NC, NS = info.num_cores, info.num_subcores
    NW = NC * NS
    assert B % (16 * NW) == 0 and D % 16 == 0
    b_per_w = B // NW
    n_chunks = b_per_w // 16
    mesh = plsc.VectorSubcoreMesh(core_axis_name="c", subcore_axis_name="s")

    @functools.partial(
        pl.kernel,
        mesh=mesh,
        compiler_params=pltpu.CompilerParams(needs_layout_passes=False),
        out_type=jax.ShapeDtypeStruct((B, 128), jnp.float32),
        scratch_types=[
            pltpu.VMEM((b_per_w + 16,), jnp.int32),
            pltpu.VMEM((b_per_w + 16,), jnp.int32),
            pltpu.VMEM((b_per_w + 16,), jnp.int32),
            pltpu.VMEM((b_per_w // 128, 128), jnp.int32),
            pltpu.VMEM((2 * _NBUF, D, 128), jnp.float32),
            pltpu.VMEM((256, 128), jnp.float32),
        ] + [pltpu.SemaphoreType.DMA] * (_NBUF + 1),
    )
    def sc_gather(tableT_hbm, ids_hbm, pos_hbm, out_hbm,
                  idx_v, s_v, ucols, pos_v, stage_v, rows_v, *sems):
        wid = lax.axis_index("s") * NC + lax.axis_index("c")
        base = wid * b_per_w
        pltpu.sync_copy(ids_hbm.at[pl.ds(base, b_per_w)],
                        idx_v.at[pl.ds(0, b_per_w)])
        pltpu.sync_copy(pos_hbm.at[pl.ds(wid * (b_per_w // 128),
                                         b_per_w // 128)], pos_v)
        iota16 = lax.broadcasted_iota(jnp.int32, (16,), 0)
        big = jnp.full((16,), jnp.int32(1 << 30))
        s_v[pl.ds(b_per_w, 16)] = big

        def chunk(c, carry):
            vv = idx_v[pl.ds(c * 16, 16)]
            t = vv // 128
            shift_idx = jnp.maximum(iota16 + c * 16 - 1, 0)
            tprev = plsc.load_gather(idx_v, [shift_idx]) // 128
            newf = (t != tprev) | ((iota16 + c * 16) == 0)
            nf = newf.astype(jnp.int32)
            cum = plsc.cumsum(nf)
            s = carry + cum - 1
            s_v[pl.ds(c * 16, 16)] = s
            plsc.store_scatter(ucols, [s], t, mask=newf)
            return carry + cum[15]

        n_unique = lax.fori_loop(0, n_chunks, chunk, 0)
        n_grp = (n_unique + _NBUF - 1) // _NBUF

        def issue_group(g):
            uvg = plsc.load_gather(ucols, [iota16 + g * _NBUF])
            par = g - (g // 2) * 2

            @pl.when(par == 0)
            def _():
                for b in range(_NBUF):
                    @pl.when(g * _NBUF + b < n_unique)
                    def _(b=b):
                        start = pl.multiple_of(uvg[b] * 128, 128)
                        pltpu.async_copy(
                            tableT_hbm.at[pl.ds(0, D), pl.ds(start, 128)],
                            stage_v.at[b], sems[b])

            @pl.when(par == 1)
            def _():
                for b in range(_NBUF):
                    @pl.when(g * _NBUF + b < n_unique)
                    def _(b=b):
                        start = pl.multiple_of(uvg[b] * 128, 128)
                        pltpu.async_copy(
                            tableT_hbm.at[pl.ds(0, D), pl.ds(start, 128)],
                            stage_v.at[_NBUF + b], sems[b])

        issue_group(0)

        def group(g, carry):
            e0, fl0 = carry
            for b in range(_NBUF):
                @pl.when(g * _NBUF + b < n_unique)
                def _(b=b):
                    pltpu.make_async_copy(
                        tableT_hbm.at[pl.ds(0, D), pl.ds(0, 128)],
                        stage_v.at[b], sems[b]).wait()

            @pl.when(g + 1 < n_grp)
            def _():
                issue_group(g + 1)

            hi = (g + 1) * _NBUF
            off = (g - (g // 2) * 2) * _NBUF

            def cond(c):
                e, fl = c
                se = plsc.load_gather(s_v, [iota16 * 0 + e])[0]
                return (e < b_per_w) & (se < hi)

            def body(c):
                e, fl = c
                s_e = plsc.load_gather(s_v, [iota16 * 0 + e])[0]
                v_e = plsc.load_gather(idx_v, [iota16 * 0 + e])[0]
                slot = iota16 * 0 + (off + s_e - g * _NBUF)
                lane = iota16 * 0 + (v_e - (v_e // 128) * 128)
                ev = iota16 * 0 + (e - (e // 256) * 256)
                for k in range(D // 16):
                    col = plsc.load_gather(
                        stage_v, [slot, iota16 + k * 16, lane])
                    plsc.store_scatter(rows_v, [ev, iota16 + k * 16], col)

                def fcond(fl2):
                    return (fl2 + 1) * 128 <= e + 1

                def fbody(fl2):
                    roff = pl.multiple_of((fl2 - (fl2 // 2) * 2) * 128, 128)
                    pltpu.async_copy(
                        rows_v.at[pl.ds(roff, 128)],
                        out_hbm.at[pos_v.at[fl2]], sems[_NBUF]).wait()
                    return fl2 + 1

                fl = lax.while_loop(fcond, fbody, fl)
                return (e + 1, fl)

            return lax.while_loop(cond, body, (e0, fl0))

        lax.fori_loop(0, n_grp, group, (0, 0))

    return sc_gather


def _mlp_block(d_model, x_ref, w1_ref, b1_ref, g_ref, be_ref, mu_ref,
               var_ref, w2_ref, b2_ref, o_ref):
    x = x_ref[...][:, :d_model]
    h = jnp.dot(x, w1_ref[...], preferred_element_type=jnp.float32)
    h = jnp.maximum(h + b1_ref[...], 0.0)
    s = g_ref[...] * lax.rsqrt(var_ref[...] + 1e-5)
    t = be_ref[...] - mu_ref[...] * s
    h = h * s + t
    o = jnp.dot(h, w2_ref[...], preferred_element_type=jnp.float32)
    o_ref[...] = jnp.maximum(o + b2_ref[...], 0.0)


def _mlp(x, D, W1, b1, gamma, beta, mu, var, W2, b2, block_b):
    B, Dw = x.shape
    H = W1.shape[1]
    grid = (B // block_b,)
    row = lambda v: v.reshape(1, H)
    rep = lambda shape: pl.BlockSpec(shape, lambda i: (0, 0))
    return pl.pallas_call(
        functools.partial(_mlp_block, D),
        grid=grid,
        in_specs=[
            pl.BlockSpec((block_b, Dw), lambda i: (i, 0)),
            rep((D, H)), rep((1, H)), rep((1, H)), rep((1, H)),
            rep((1, H)), rep((1, H)), rep((H, H)), rep((1, H)),
        ],
        out_specs=pl.BlockSpec((block_b, H), lambda i: (i, 0)),
        out_shape=jax.ShapeDtypeStruct((B, H), jnp.float32),
    )(x, W1, row(b1), row(gamma), row(beta), row(mu), row(var), W2, row(b2))


def kernel(target_ids, emb_table, W1, b1, gamma, beta, running_mean,
           running_var, W2, b2):
    V, D = emb_table.shape
    B = target_ids.shape[0]
    ids = target_ids.astype(jnp.int32)
    ids_s, pos = lax.sort_key_val(ids, lax.iota(jnp.int32, B))
    pos2 = pos.reshape(B // 128, 128)
    xw = _make_sc_gather(V, D, B)(emb_table.T, ids_s, pos2)
    return _mlp(xw, D, W1, b1, gamma, beta, running_mean, running_var, W2, b2,
                block_b=2048)
